# baseline (speedup 1.0000x reference)
import functools

import jax
import jax.numpy as jnp
from jax import lax
from jax.experimental import pallas as pl
from jax.experimental.pallas import tpu as pltpu
from jax.experimental.pallas import tpu_sc as plsc

N = 10000
E = 320000
D = 128
C = 64
EPS = 1e-5

NC = 2
NS = 16
NW = NC * NS

CH = 128
NCHT = 160
NCHP = 80
E_PAD = NS * CH * NCHT
EPT = E_PAD // NW
DEG_NCH = EPT // CH
RPT = 632
N_ACC = NS * RPT
DW = 16


@functools.cache
def _sc_kernels():
    mesh = plsc.VectorSubcoreMesh(
        core_axis_name="c", subcore_axis_name="s", num_cores=NC, num_subcores=NS)

    @functools.partial(
        pl.kernel,
        out_type=jax.ShapeDtypeStruct((NC * N_ACC,), jnp.float32),
        mesh=mesh,
        scratch_types=[
            pltpu.VMEM((CH,), jnp.int32),
            pltpu.VMEM((CH,), jnp.float32),
            pltpu.VMEM((RPT,), jnp.float32),
            pltpu.VMEM_SHARED((N_ACC,), jnp.float32),
        ],
    )
    def deg_sc(dst_hbm, ones_hbm, zrow_hbm, out_hbm, dst_v, ones_v, stage_v, acc_sh):
        c = lax.axis_index("c")
        s = lax.axis_index("s")
        w = c * NS + s
        pltpu.sync_copy(ones_hbm, ones_v)
        pltpu.sync_copy(zrow_hbm, stage_v)
        pltpu.sync_copy(stage_v, acc_sh.at[pl.ds(s * RPT, RPT)])
        plsc.subcore_barrier()
        base = w * EPT

        def body(k, carry):
            pltpu.sync_copy(dst_hbm.at[pl.ds(base + k * CH, CH)], dst_v)
            pltpu.sync_copy(ones_v, acc_sh.at[dst_v], add=True)
            return carry

        lax.fori_loop(0, DEG_NCH, body, 0)
        plsc.subcore_barrier()
        pltpu.sync_copy(acc_sh.at[pl.ds(s * RPT, RPT)], stage_v)
        pltpu.sync_copy(stage_v, out_hbm.at[pl.ds(c * N_ACC + s * RPT, RPT)])

    mesh1 = plsc.VectorSubcoreMesh(
        core_axis_name="c", subcore_axis_name="s", num_cores=1, num_subcores=NS)

    @functools.partial(
        pl.kernel,
        out_type=jax.ShapeDtypeStruct((N_ACC, D), jnp.float32),
        mesh=mesh1,
        scratch_types=[
            pltpu.VMEM((CH,), jnp.int32),
            pltpu.VMEM((CH,), jnp.int32),
            pltpu.VMEM((NCHP, CH), jnp.int32),
            pltpu.VMEM((CH, D), jnp.float32),
            pltpu.VMEM((CH, D), jnp.float32),
            pltpu.VMEM((8, D), jnp.float32),
            pltpu.VMEM_SHARED((N_ACC, D), jnp.float32),
            pltpu.SemaphoreType.DMA,
            pltpu.SemaphoreType.DMA,
        ],
    )
    def prop_sc(z_hbm, src_hbm, dst3_hbm, zrows_hbm, out_hbm,
                src_a, src_b, dst_all, rows_a, rows_b, stage_v, acc_sh,
                sem_a, sem_b):
        s = lax.axis_index("s")
        pltpu.sync_copy(zrows_hbm, stage_v)

        def zbody(j, carry):
            pltpu.sync_copy(stage_v, acc_sh.at[pl.ds(s * RPT + j * 8, 8)])
            return carry

        lax.fori_loop(0, RPT // 8, zbody, 0)
        plsc.subcore_barrier()

        def pipe(nch, base_e, cb):
            pltpu.sync_copy(dst3_hbm.at[pl.ds(cb, nch)], dst_all.at[pl.ds(0, nch)])
            pltpu.sync_copy(src_hbm.at[pl.ds(base_e, CH)], src_a)
            pltpu.async_copy(z_hbm.at[src_a], rows_a, sem_a)
            n2 = nch // 2

            def body(i, carry):
                k0 = 2 * i
                pltpu.sync_copy(src_hbm.at[pl.ds(base_e + (k0 + 1) * CH, CH)], src_b)
                pltpu.async_copy(z_hbm.at[src_b], rows_b, sem_b)
                pltpu.make_async_copy(z_hbm.at[src_a], rows_a, sem_a).wait()
                pltpu.sync_copy(rows_a, acc_sh.at[dst_all.at[k0]], add=True)

                @pl.when(i + 1 < n2)
                def _():
                    pltpu.sync_copy(src_hbm.at[pl.ds(base_e + (k0 + 2) * CH, CH)], src_a)
                    pltpu.async_copy(z_hbm.at[src_a], rows_a, sem_a)

                pltpu.make_async_copy(z_hbm.at[src_b], rows_b, sem_b).wait()
                pltpu.sync_copy(rows_b, acc_sh.at[dst_all.at[k0 + 1]], add=True)
                return carry

            lax.fori_loop(0, n2, body, 0)

        pipe(NCHP, s * NCHT * CH, s * NCHT)
        pipe(NCHP, (s * NCHT + NCHP) * CH, s * NCHT + NCHP)

        plsc.subcore_barrier()
        pltpu.sync_copy(
            acc_sh.at[pl.ds(s * RPT, RPT)],
            out_hbm.at[pl.ds(s * RPT, RPT)],
        )

    return deg_sc, prop_sc


def _bn_relu(h, g, be):
    mu = jnp.mean(h, axis=0, keepdims=True)
    var = jnp.mean((h - mu) * (h - mu), axis=0, keepdims=True)
    return jnp.maximum((h - mu) * lax.rsqrt(var + EPS) * g + be, 0.0)


def _tc_in_body(x_ref, w_ref, b_ref, g_ref, be_ref, w0_ref, d0_ref, d1_ref,
                dinv_ref, z0_ref):
    h = jnp.dot(x_ref[...], w_ref[...], preferred_element_type=jnp.float32)
    h = _bn_relu(h + b_ref[...], g_ref[...], be_ref[...])
    dinv = lax.rsqrt(d0_ref[...] + d1_ref[...] + 1.0)
    dinv_ref[...] = dinv
    z0_ref[...] = jnp.dot(h, w0_ref[...], preferred_element_type=jnp.float32) * dinv


def _tc_mid_body(p_ref, z_ref, dinv_ref, b_ref, g_ref, be_ref, w_ref,
                 zn_ref):
    dinv = dinv_ref[...]
    acc = p_ref[:N, :] + z_ref[...]
    h = _bn_relu(acc * dinv + b_ref[...], g_ref[...], be_ref[...])
    zn_ref[...] = jnp.dot(h, w_ref[...], preferred_element_type=jnp.float32) * dinv


def _tc_out_body(p_ref, z_ref, dinv_ref, b_ref, wout_ref, bout_ref, y_ref):
    acc = p_ref[:N, :] + z_ref[...]
    h = acc * dinv_ref[...] + b_ref[...]
    y_ref[...] = jnp.dot(h, wout_ref[...], preferred_element_type=jnp.float32) + bout_ref[...]


_f32 = jnp.float32

_tc_in = pl.pallas_call(
    _tc_in_body,
    out_shape=(jax.ShapeDtypeStruct((N, 1), _f32),
               jax.ShapeDtypeStruct((N, D), _f32)),
)

_tc_mid = pl.pallas_call(
    _tc_mid_body,
    out_shape=jax.ShapeDtypeStruct((N, D), _f32),
)

_tc_out = pl.pallas_call(
    _tc_out_body,
    out_shape=jax.ShapeDtypeStruct((N, C), _f32),
)


def kernel(x, edge_index, W_in, b_in, g_in, be_in, W0, b0, g0, be0,
           W1, b1, g1, be1, W2, b2, W_out, b_out):
    pad = E_PAD - E
    srcp = jnp.concatenate([edge_index[0], jnp.zeros((pad,), edge_index.dtype)])
    dstp = jnp.concatenate([edge_index[1], jnp.full((pad,), N, edge_index.dtype)])

    ones1 = jnp.ones((CH,), _f32)
    z1d = jnp.zeros((RPT,), _f32)
    zrows = jnp.zeros((8, D), _f32)

    _deg_sc, _prop_sc = _sc_kernels()
    degf = _deg_sc(dstp, ones1, z1d)
    d0 = degf[:N].reshape(N, 1)
    d1 = degf[N_ACC:N_ACC + N].reshape(N, 1)

    dinv, z0 = _tc_in(x, W_in, b_in.reshape(1, D), g_in.reshape(1, D),
                      be_in.reshape(1, D), W0, d0, d1)

    dst3 = dstp.reshape(E_PAD // CH, CH)
    p0 = _prop_sc(z0, srcp, dst3, zrows)
    z1 = _tc_mid(p0, z0, dinv, b0.reshape(1, D), g0.reshape(1, D),
                 be0.reshape(1, D), W1)
    p1 = _prop_sc(z1, srcp, dst3, zrows)
    z2 = _tc_mid(p1, z1, dinv, b1.reshape(1, D), g1.reshape(1, D),
                 be1.reshape(1, D), W2)
    p2 = _prop_sc(z2, srcp, dst3, zrows)
    return _tc_out(p2, z2, dinv, b2.reshape(1, D), W_out, b_out.reshape(1, C))


# speedup vs baseline: 3.2241x; 3.2241x over previous
import functools

import jax
import jax.numpy as jnp
from jax import lax
from jax.experimental import pallas as pl
from jax.experimental.pallas import tpu as pltpu
from jax.experimental.pallas import tpu_sc as plsc

N = 10000
E = 320000
D = 128
C = 64
EPS = 1e-5

NC = 2
NS = 16
NW = NC * NS

CH = 128
NCHT = 80
E_PAD = NW * CH * NCHT
EPT = E_PAD // NW
DEG_NCH = EPT // CH
RPT = 640
N_ACC = NS * RPT
DW = 16


@functools.cache
def _sc_kernels():
    mesh = plsc.VectorSubcoreMesh(
        core_axis_name="c", subcore_axis_name="s", num_cores=NC, num_subcores=NS)

    @functools.partial(
        pl.kernel,
        out_type=jax.ShapeDtypeStruct((NC * N_ACC,), jnp.float32),
        mesh=mesh,
        scratch_types=[
            pltpu.VMEM((CH,), jnp.int32),
            pltpu.VMEM((CH,), jnp.float32),
            pltpu.VMEM((RPT,), jnp.float32),
            pltpu.VMEM_SHARED((N_ACC,), jnp.float32),
        ],
    )
    def deg_sc(dst_hbm, ones_hbm, zrow_hbm, out_hbm, dst_v, ones_v, stage_v, acc_sh):
        c = lax.axis_index("c")
        s = lax.axis_index("s")
        w = c * NS + s
        pltpu.sync_copy(ones_hbm, ones_v)
        pltpu.sync_copy(zrow_hbm, stage_v)
        pltpu.sync_copy(stage_v, acc_sh.at[pl.ds(s * RPT, RPT)])
        plsc.subcore_barrier()
        base = w * EPT

        def body(k, carry):
            pltpu.sync_copy(dst_hbm.at[pl.ds(base + k * CH, CH)], dst_v)
            pltpu.sync_copy(ones_v, acc_sh.at[dst_v], add=True)
            return carry

        lax.fori_loop(0, DEG_NCH, body, 0)
        plsc.subcore_barrier()
        pltpu.sync_copy(acc_sh.at[pl.ds(s * RPT, RPT)], stage_v)
        pltpu.sync_copy(stage_v, out_hbm.at[pl.ds(c * N_ACC + s * RPT, RPT)])

    @functools.partial(
        pl.kernel,
        out_type=jax.ShapeDtypeStruct((NC * N_ACC, D), jnp.float32),
        mesh=mesh,
        scratch_types=[
            pltpu.VMEM((CH,), jnp.int32),
            pltpu.VMEM((CH,), jnp.int32),
            pltpu.VMEM((NCHT, CH), jnp.int32),
            pltpu.VMEM((CH, D), jnp.float32),
            pltpu.VMEM((CH, D), jnp.float32),
            pltpu.VMEM((8, D), jnp.float32),
            pltpu.VMEM_SHARED((N_ACC, D), jnp.float32),
            pltpu.SemaphoreType.DMA,
            pltpu.SemaphoreType.DMA,
        ],
    )
    def prop_sc(z_hbm, src_hbm, dst3_hbm, zrows_hbm, out_hbm,
                src_a, src_b, dst_all, rows_a, rows_b, stage_v, acc_sh,
                sem_a, sem_b):
        c = lax.axis_index("c")
        s = lax.axis_index("s")
        w = c * NS + s
        pltpu.sync_copy(zrows_hbm, stage_v)

        def zbody(j, carry):
            pltpu.sync_copy(stage_v, acc_sh.at[pl.ds(s * RPT + j * 8, 8)])
            return carry

        lax.fori_loop(0, RPT // 8, zbody, 0)
        plsc.subcore_barrier()

        def pipe(nch, base_e, cb):
            pltpu.sync_copy(dst3_hbm.at[pl.ds(cb, nch)], dst_all.at[pl.ds(0, nch)])
            pltpu.sync_copy(src_hbm.at[pl.ds(base_e, CH)], src_a)
            pltpu.async_copy(z_hbm.at[src_a], rows_a, sem_a)
            n2 = nch // 2

            def body(i, carry):
                k0 = 2 * i
                pltpu.sync_copy(src_hbm.at[pl.ds(base_e + (k0 + 1) * CH, CH)], src_b)
                pltpu.async_copy(z_hbm.at[src_b], rows_b, sem_b)
                pltpu.make_async_copy(z_hbm.at[src_a], rows_a, sem_a).wait()
                pltpu.sync_copy(rows_a, acc_sh.at[dst_all.at[k0]], add=True)

                @pl.when(i + 1 < n2)
                def _():
                    pltpu.sync_copy(src_hbm.at[pl.ds(base_e + (k0 + 2) * CH, CH)], src_a)
                    pltpu.async_copy(z_hbm.at[src_a], rows_a, sem_a)

                pltpu.make_async_copy(z_hbm.at[src_b], rows_b, sem_b).wait()
                pltpu.sync_copy(rows_b, acc_sh.at[dst_all.at[k0 + 1]], add=True)
                return carry

            lax.fori_loop(0, n2, body, 0)

        pipe(NCHT, w * NCHT * CH, w * NCHT)

        plsc.subcore_barrier()
        pltpu.sync_copy(
            acc_sh.at[pl.ds(s * RPT, RPT)],
            out_hbm.at[pl.ds(c * N_ACC + s * RPT, RPT)],
        )

    return deg_sc, prop_sc


def _bn_relu(h, g, be):
    mu = jnp.mean(h, axis=0, keepdims=True)
    var = jnp.mean((h - mu) * (h - mu), axis=0, keepdims=True)
    return jnp.maximum((h - mu) * lax.rsqrt(var + EPS) * g + be, 0.0)


def _tc_in_body(x_ref, w_ref, b_ref, g_ref, be_ref, w0_ref, d0_ref, d1_ref,
                dinv_ref, z0_ref):
    h = jnp.dot(x_ref[...], w_ref[...], preferred_element_type=jnp.float32)
    h = _bn_relu(h + b_ref[...], g_ref[...], be_ref[...])
    dinv = lax.rsqrt(d0_ref[...] + d1_ref[...] + 1.0)
    dinv_ref[...] = dinv
    z0_ref[...] = jnp.dot(h, w0_ref[...], preferred_element_type=jnp.float32) * dinv


def _tc_mid_body(p_ref, z_ref, dinv_ref, b_ref, g_ref, be_ref, w_ref,
                 zn_ref):
    dinv = dinv_ref[...]
    acc = p_ref[:N, :] + p_ref[N_ACC:N_ACC + N, :] + z_ref[...]
    h = _bn_relu(acc * dinv + b_ref[...], g_ref[...], be_ref[...])
    zn_ref[...] = jnp.dot(h, w_ref[...], preferred_element_type=jnp.float32) * dinv


def _tc_out_body(p_ref, z_ref, dinv_ref, b_ref, wout_ref, bout_ref, y_ref):
    acc = p_ref[:N, :] + p_ref[N_ACC:N_ACC + N, :] + z_ref[...]
    h = acc * dinv_ref[...] + b_ref[...]
    y_ref[...] = jnp.dot(h, wout_ref[...], preferred_element_type=jnp.float32) + bout_ref[...]


_f32 = jnp.float32

_tc_in = pl.pallas_call(
    _tc_in_body,
    out_shape=(jax.ShapeDtypeStruct((N, 1), _f32),
               jax.ShapeDtypeStruct((N, D), _f32)),
)

_tc_mid = pl.pallas_call(
    _tc_mid_body,
    out_shape=jax.ShapeDtypeStruct((N, D), _f32),
)

_tc_out = pl.pallas_call(
    _tc_out_body,
    out_shape=jax.ShapeDtypeStruct((N, C), _f32),
)


def kernel(x, edge_index, W_in, b_in, g_in, be_in, W0, b0, g0, be0,
           W1, b1, g1, be1, W2, b2, W_out, b_out):
    pad = E_PAD - E
    pad_src = (jnp.arange(pad, dtype=edge_index.dtype) * 997) % N
    pad_dst = N + (jnp.arange(pad, dtype=edge_index.dtype) % (N_ACC - N))
    srcp = jnp.concatenate([edge_index[0], pad_src])
    dstp = jnp.concatenate([edge_index[1], pad_dst])

    ones1 = jnp.ones((CH,), _f32)
    z1d = jnp.zeros((RPT,), _f32)
    zrows = jnp.zeros((8, D), _f32)

    _deg_sc, _prop_sc = _sc_kernels()
    degf = _deg_sc(dstp, ones1, z1d)
    d0 = degf[:N].reshape(N, 1)
    d1 = degf[N_ACC:N_ACC + N].reshape(N, 1)

    dinv, z0 = _tc_in(x, W_in, b_in.reshape(1, D), g_in.reshape(1, D),
                      be_in.reshape(1, D), W0, d0, d1)

    dst3 = dstp.reshape(E_PAD // CH, CH)
    p0 = _prop_sc(z0, srcp, dst3, zrows)
    z1 = _tc_mid(p0, z0, dinv, b0.reshape(1, D), g0.reshape(1, D),
                 be0.reshape(1, D), W1)
    p1 = _prop_sc(z1, srcp, dst3, zrows)
    z2 = _tc_mid(p1, z1, dinv, b1.reshape(1, D), g1.reshape(1, D),
                 be1.reshape(1, D), W2)
    p2 = _prop_sc(z2, srcp, dst3, zrows)
    return _tc_out(p2, z2, dinv, b2.reshape(1, D), W_out, b_out.reshape(1, C))


# speedup vs baseline: 3.5343x; 1.0962x over previous
import functools

import jax
import jax.numpy as jnp
from jax import lax
from jax.experimental import pallas as pl
from jax.experimental.pallas import tpu as pltpu
from jax.experimental.pallas import tpu_sc as plsc

N = 10000
E = 320000
D = 128
C = 64
EPS = 1e-5

NC = 2
NS = 16
NW = NC * NS

CH = 128
NCHT = 80
E_PAD = NW * CH * NCHT
EPT = E_PAD // NW
DEG_NCH = EPT // CH
RPT = 640
N_ACC = NS * RPT
DW = 16


@functools.cache
def _sc_kernels():
    mesh = plsc.VectorSubcoreMesh(
        core_axis_name="c", subcore_axis_name="s", num_cores=NC, num_subcores=NS)

    @functools.partial(
        pl.kernel,
        out_type=jax.ShapeDtypeStruct((NC * N_ACC,), jnp.float32),
        mesh=mesh,
        scratch_types=[
            pltpu.VMEM((DEG_NCH, CH), jnp.int32),
            pltpu.VMEM((CH,), jnp.float32),
            pltpu.VMEM((RPT,), jnp.float32),
            pltpu.VMEM_SHARED((N_ACC,), jnp.float32),
        ],
    )
    def deg_sc(dst3_hbm, ones_hbm, zrow_hbm, out_hbm, dst_all, ones_v, stage_v, acc_sh):
        c = lax.axis_index("c")
        s = lax.axis_index("s")
        w = c * NS + s
        pltpu.sync_copy(dst3_hbm.at[pl.ds(w * DEG_NCH, DEG_NCH)], dst_all)
        pltpu.sync_copy(ones_hbm, ones_v)
        pltpu.sync_copy(zrow_hbm, stage_v)
        pltpu.sync_copy(stage_v, acc_sh.at[pl.ds(s * RPT, RPT)])
        plsc.subcore_barrier()

        def body(k, carry):
            pltpu.sync_copy(ones_v, acc_sh.at[dst_all.at[k]], add=True)
            return carry

        lax.fori_loop(0, DEG_NCH, body, 0)
        plsc.subcore_barrier()
        pltpu.sync_copy(acc_sh.at[pl.ds(s * RPT, RPT)], stage_v)
        pltpu.sync_copy(stage_v, out_hbm.at[pl.ds(c * N_ACC + s * RPT, RPT)])

    @functools.partial(
        pl.kernel,
        out_type=jax.ShapeDtypeStruct((NC * N_ACC, D), jnp.float32),
        mesh=mesh,
        scratch_types=[
            pltpu.VMEM((CH,), jnp.int32),
            pltpu.VMEM((CH,), jnp.int32),
            pltpu.VMEM((NCHT, CH), jnp.int32),
            pltpu.VMEM((CH, D), jnp.float32),
            pltpu.VMEM((CH, D), jnp.float32),
            pltpu.VMEM((32, D), jnp.float32),
            pltpu.VMEM_SHARED((N_ACC, D), jnp.float32),
            pltpu.SemaphoreType.DMA,
            pltpu.SemaphoreType.DMA,
        ],
    )
    def prop_sc(z_hbm, src_hbm, dst3_hbm, zrows_hbm, out_hbm,
                src_a, src_b, dst_all, rows_a, rows_b, stage_v, acc_sh,
                sem_a, sem_b):
        c = lax.axis_index("c")
        s = lax.axis_index("s")
        w = c * NS + s
        base_e = w * NCHT * CH
        pltpu.sync_copy(dst3_hbm.at[pl.ds(w * NCHT, NCHT)], dst_all)
        pltpu.sync_copy(src_hbm.at[pl.ds(base_e, CH)], src_a)
        pltpu.async_copy(z_hbm.at[src_a], rows_a, sem_a)
        pltpu.sync_copy(zrows_hbm, stage_v)

        def zbody(j, carry):
            pltpu.sync_copy(stage_v, acc_sh.at[pl.ds(s * RPT + j * 32, 32)])
            return carry

        lax.fori_loop(0, RPT // 32, zbody, 0)
        plsc.subcore_barrier()

        def pipe(nch):
            n2 = nch // 2

            def body(i, carry):
                k0 = 2 * i
                pltpu.sync_copy(src_hbm.at[pl.ds(base_e + (k0 + 1) * CH, CH)], src_b)
                pltpu.async_copy(z_hbm.at[src_b], rows_b, sem_b)
                pltpu.make_async_copy(z_hbm.at[src_a], rows_a, sem_a).wait()
                pltpu.sync_copy(rows_a, acc_sh.at[dst_all.at[k0]], add=True)

                @pl.when(i + 1 < n2)
                def _():
                    pltpu.sync_copy(src_hbm.at[pl.ds(base_e + (k0 + 2) * CH, CH)], src_a)
                    pltpu.async_copy(z_hbm.at[src_a], rows_a, sem_a)

                pltpu.make_async_copy(z_hbm.at[src_b], rows_b, sem_b).wait()
                pltpu.sync_copy(rows_b, acc_sh.at[dst_all.at[k0 + 1]], add=True)
                return carry

            lax.fori_loop(0, n2, body, 0)

        pipe(NCHT)

        plsc.subcore_barrier()
        pltpu.sync_copy(
            acc_sh.at[pl.ds(s * RPT, RPT)],
            out_hbm.at[pl.ds(c * N_ACC + s * RPT, RPT)],
        )

    return deg_sc, prop_sc


def _bn_relu(h, g, be):
    mu = jnp.mean(h, axis=0, keepdims=True)
    var = jnp.mean((h - mu) * (h - mu), axis=0, keepdims=True)
    return jnp.maximum((h - mu) * lax.rsqrt(var + EPS) * g + be, 0.0)


def _tc_in_body(x_ref, w_ref, b_ref, g_ref, be_ref, w0_ref, d0_ref, d1_ref,
                dinv_ref, z0_ref):
    h = jnp.dot(x_ref[...], w_ref[...], preferred_element_type=jnp.float32)
    h = _bn_relu(h + b_ref[...], g_ref[...], be_ref[...])
    dinv = lax.rsqrt(d0_ref[...] + d1_ref[...] + 1.0)
    dinv_ref[...] = dinv
    z0_ref[...] = jnp.dot(h, w0_ref[...], preferred_element_type=jnp.float32) * dinv


def _tc_mid_body(p_ref, z_ref, dinv_ref, b_ref, g_ref, be_ref, w_ref,
                 zn_ref):
    dinv = dinv_ref[...]
    acc = p_ref[:N, :] + p_ref[N_ACC:N_ACC + N, :] + z_ref[...]
    h = _bn_relu(acc * dinv + b_ref[...], g_ref[...], be_ref[...])
    zn_ref[...] = jnp.dot(h, w_ref[...], preferred_element_type=jnp.float32) * dinv


def _tc_out_body(p_ref, z_ref, dinv_ref, b_ref, wout_ref, bout_ref, y_ref):
    acc = p_ref[:N, :] + p_ref[N_ACC:N_ACC + N, :] + z_ref[...]
    h = acc * dinv_ref[...] + b_ref[...]
    y_ref[...] = jnp.dot(h, wout_ref[...], preferred_element_type=jnp.float32) + bout_ref[...]


_f32 = jnp.float32

_tc_in = pl.pallas_call(
    _tc_in_body,
    out_shape=(jax.ShapeDtypeStruct((N, 1), _f32),
               jax.ShapeDtypeStruct((N, D), _f32)),
)

_tc_mid = pl.pallas_call(
    _tc_mid_body,
    out_shape=jax.ShapeDtypeStruct((N, D), _f32),
)

_tc_out = pl.pallas_call(
    _tc_out_body,
    out_shape=jax.ShapeDtypeStruct((N, C), _f32),
)


def kernel(x, edge_index, W_in, b_in, g_in, be_in, W0, b0, g0, be0,
           W1, b1, g1, be1, W2, b2, W_out, b_out):
    pad = E_PAD - E
    pad_src = (jnp.arange(pad, dtype=edge_index.dtype) * 997) % N
    pad_dst = N + (jnp.arange(pad, dtype=edge_index.dtype) % (N_ACC - N))
    srcp = jnp.concatenate([edge_index[0], pad_src])
    dstp = jnp.concatenate([edge_index[1], pad_dst])

    ones1 = jnp.ones((CH,), _f32)
    z1d = jnp.zeros((RPT,), _f32)
    zrows = jnp.zeros((32, D), _f32)

    dst3i = dstp.reshape(E_PAD // CH, CH)
    _deg_sc, _prop_sc = _sc_kernels()
    degf = _deg_sc(dst3i, ones1, z1d)
    d0 = degf[:N].reshape(N, 1)
    d1 = degf[N_ACC:N_ACC + N].reshape(N, 1)

    dinv, z0 = _tc_in(x, W_in, b_in.reshape(1, D), g_in.reshape(1, D),
                      be_in.reshape(1, D), W0, d0, d1)

    p0 = _prop_sc(z0, srcp, dst3i, zrows)
    z1 = _tc_mid(p0, z0, dinv, b0.reshape(1, D), g0.reshape(1, D),
                 be0.reshape(1, D), W1)
    p1 = _prop_sc(z1, srcp, dst3i, zrows)
    z2 = _tc_mid(p1, z1, dinv, b1.reshape(1, D), g1.reshape(1, D),
                 be1.reshape(1, D), W2)
    p2 = _prop_sc(z2, srcp, dst3i, zrows)
    return _tc_out(p2, z2, dinv, b2.reshape(1, D), W_out, b_out.reshape(1, C))
